# baseline (speedup 1.0000x reference)
import jax
import jax.numpy as jnp
from jax.experimental import pallas as pl
from jax.experimental.pallas import tpu as pltpu

_ALPHA = 0.25
_IGNORE_INDEX = -100

_ROWS = 1024
_COLS = 100000
_PCOLS = _COLS // 2
_RBLK = 256
_CSUB = 4096
_CHUNKS = _CSUB // 128
_NJ = _PCOLS // _CSUB + 1

_LOG2E = 1.4426950408889634
_LN2 = 0.6931471805599453


def _focal_kernel(x_ref, tgt_ref, out_ref, m_ref, s_ref, t_ref):
    i = pl.program_id(0)
    j = pl.program_id(1)

    @pl.when(j == 0)
    def _init():
        m_ref[...] = jnp.full((_RBLK, 128), -jnp.inf, jnp.float32)
        s_ref[...] = jnp.zeros((_RBLK, 128), jnp.float32)
        t_ref[...] = jnp.zeros((_RBLK, 128), jnp.float32)

    tgt = tgt_ref[...]
    lane = jax.lax.broadcasted_iota(jnp.int32, (_RBLK, 128), 1)
    rel_ptgt = (tgt >> 1) - j * _CSUB
    rel_pend = _PCOLS - j * _CSUB

    rel_ptgt_b = jnp.broadcast_to(rel_ptgt, (_RBLK, 128))
    odd_b = jnp.broadcast_to((tgt & 1) == 1, (_RBLK, 128))
    rel_pend_b = jnp.broadcast_to(rel_pend, (_RBLK, 128))

    def unpack(c, masked):
        pb = x_ref[:, c * 128:(c + 1) * 128]
        bits = jax.lax.bitcast_convert_type(pb, jnp.uint32)
        xe = jax.lax.bitcast_convert_type(
            bits & jnp.uint32(0xFFFF0000), jnp.float32
        )
        xo = jax.lax.bitcast_convert_type(bits << 16, jnp.float32)
        if masked:
            valid = lane + c * 128 < rel_pend_b
            xe = jnp.where(valid, xe, -jnp.inf)
            xo = jnp.where(valid, xo, -jnp.inf)
        return xe, xo

    def process(chunks):
        m_old = m_ref[...]
        s = s_ref[...]
        t = t_ref[...]
        for g in range(0, len(chunks), 4):
            group = chunks[g:g + 4]
            xs = []
            for c, masked in group:
                xe, xo = unpack(c, masked)
                xs.append((c, xe, xo))
            bm = jnp.maximum(xs[0][1], xs[0][2])
            for _, xe, xo in xs[1:]:
                bm = jnp.maximum(bm, jnp.maximum(xe, xo))
            m_new = jnp.maximum(m_old, bm)
            s = s * jnp.exp2((m_old - m_new) * _LOG2E)
            eg = None
            for c, xe, xo in xs:
                e = jnp.exp2((xe - m_new) * _LOG2E) + jnp.exp2(
                    (xo - m_new) * _LOG2E
                )
                eg = e if eg is None else eg + e
                t = jnp.where(
                    lane + c * 128 == rel_ptgt_b,
                    jnp.where(odd_b, xo, xe),
                    t,
                )
            s = s + eg
            m_old = m_new
        m_ref[...] = m_old
        s_ref[...] = s
        t_ref[...] = t
        return m_old, s, t

    is_last = j == _NJ - 1

    @pl.when(jnp.logical_not(is_last))
    def _full_step():
        process([(c, False) for c in range(_CHUNKS)])

    @pl.when(is_last)
    def _last_step():
        base = (_NJ - 1) * _CSUB
        chunks = []
        for c in range(_CHUNKS):
            start = base + c * 128
            if start + 128 <= _PCOLS:
                chunks.append((c, False))
            elif start < _PCOLS:
                chunks.append((c, True))
        m_lane, s_lane, t_lane = process(chunks)
        m_row = jnp.max(m_lane, axis=1, keepdims=True)
        s_row = jnp.sum(
            s_lane * jnp.exp2((m_lane - m_row) * _LOG2E),
            axis=1,
            keepdims=True,
        )
        t_row = jnp.sum(t_lane, axis=1, keepdims=True)
        nll = m_row + _LN2 * jnp.log2(s_row) - t_row
        loss = jnp.where(tgt == _IGNORE_INDEX, 0.0, nll)
        pt = jnp.exp(-loss)
        fl = _ALPHA * (1.0 - pt) * (1.0 - pt) * loss
        partial = jnp.sum(fl) * (1.0 / _ROWS)

        @pl.when(i == 0)
        def _zero():
            out_ref[0, 0] = 0.0

        out_ref[0, 0] += partial


def kernel(input, target):
    xb = input.astype(jnp.bfloat16)
    u = jax.lax.bitcast_convert_type(xb, jnp.uint16)
    u3 = u.reshape(_ROWS, _PCOLS, 2)
    packed_bits = (
        u3[..., 0].astype(jnp.uint32) << 16
    ) | u3[..., 1].astype(jnp.uint32)
    packed = jax.lax.bitcast_convert_type(packed_bits, jnp.float32)

    tgt2d = target.astype(jnp.int32).reshape(_ROWS, 1)
    out = pl.pallas_call(
        _focal_kernel,
        grid=(_ROWS // _RBLK, _NJ),
        in_specs=[
            pl.BlockSpec((_RBLK, _CSUB), lambda i, j: (i, j)),
            pl.BlockSpec((_RBLK, 1), lambda i, j: (i, 0)),
        ],
        out_specs=pl.BlockSpec(
            (1, 1), lambda i, j: (0, 0), memory_space=pltpu.SMEM
        ),
        out_shape=jax.ShapeDtypeStruct((1, 1), jnp.float32),
        scratch_shapes=[
            pltpu.VMEM((_RBLK, 128), jnp.float32),
            pltpu.VMEM((_RBLK, 128), jnp.float32),
            pltpu.VMEM((_RBLK, 128), jnp.float32),
        ],
    )(packed, tgt2d)
    return out[0, 0]


# speedup vs baseline: 3.9892x; 3.9892x over previous
import jax
import jax.numpy as jnp
from jax.experimental import pallas as pl
from jax.experimental.pallas import tpu as pltpu

_ALPHA = 0.25
_IGNORE_INDEX = -100

_ROWS = 1024
_COLS = 100000
_PCOLS = _COLS // 2
_RBLK = 256
_CSUB = 4096
_CHUNKS = _CSUB // 128
_NJ = _PCOLS // _CSUB + 1

_LOG2E = 1.4426950408889634
_LN2 = 0.6931471805599453


def _focal_kernel(x_ref, tgt_ref, out_ref, m_ref, s_ref, t_ref):
    i = pl.program_id(0)
    j = pl.program_id(1)

    @pl.when(j == 0)
    def _init():
        m_ref[...] = jnp.full((_RBLK, 128), -jnp.inf, jnp.float32)
        s_ref[...] = jnp.zeros((_RBLK, 128), jnp.float32)
        t_ref[...] = jnp.zeros((_RBLK, 128), jnp.float32)

    tgt = tgt_ref[...]
    lane = jax.lax.broadcasted_iota(jnp.int32, (_RBLK, 128), 1)
    phys_tgt = jnp.where(tgt >= _PCOLS, tgt - _PCOLS, tgt)
    rel_ptgt = phys_tgt - j * _CSUB
    rel_pend = _PCOLS - j * _CSUB

    rel_ptgt_b = jnp.broadcast_to(rel_ptgt, (_RBLK, 128))
    hi_b = jnp.broadcast_to(tgt < _PCOLS, (_RBLK, 128))
    rel_pend_b = jnp.broadcast_to(rel_pend, (_RBLK, 128))

    def unpack(c, masked):
        pb = x_ref[:, c * 128:(c + 1) * 128]
        bits = jax.lax.bitcast_convert_type(pb, jnp.uint32)
        xe = jax.lax.bitcast_convert_type(
            bits & jnp.uint32(0xFFFF0000), jnp.float32
        )
        xo = jax.lax.bitcast_convert_type(bits << 16, jnp.float32)
        if masked:
            valid = lane + c * 128 < rel_pend_b
            xe = jnp.where(valid, xe, -jnp.inf)
            xo = jnp.where(valid, xo, -jnp.inf)
        return xe, xo

    def process(chunks):
        m_old = m_ref[...]
        s = s_ref[...]
        t = t_ref[...]
        for g in range(0, len(chunks), 4):
            group = chunks[g:g + 4]
            xs = []
            for c, masked in group:
                xe, xo = unpack(c, masked)
                xs.append((c, xe, xo))
            bm = jnp.maximum(xs[0][1], xs[0][2])
            for _, xe, xo in xs[1:]:
                bm = jnp.maximum(bm, jnp.maximum(xe, xo))
            m_new = jnp.maximum(m_old, bm)
            s = s * jnp.exp2((m_old - m_new) * _LOG2E)
            eg = None
            for c, xe, xo in xs:
                e = jnp.exp2((xe - m_new) * _LOG2E) + jnp.exp2(
                    (xo - m_new) * _LOG2E
                )
                eg = e if eg is None else eg + e
                t = jnp.where(
                    lane + c * 128 == rel_ptgt_b,
                    jnp.where(hi_b, xe, xo),
                    t,
                )
            s = s + eg
            m_old = m_new
        m_ref[...] = m_old
        s_ref[...] = s
        t_ref[...] = t
        return m_old, s, t

    is_last = j == _NJ - 1

    @pl.when(jnp.logical_not(is_last))
    def _full_step():
        process([(c, False) for c in range(_CHUNKS)])

    @pl.when(is_last)
    def _last_step():
        base = (_NJ - 1) * _CSUB
        chunks = []
        for c in range(_CHUNKS):
            start = base + c * 128
            if start + 128 <= _PCOLS:
                chunks.append((c, False))
            elif start < _PCOLS:
                chunks.append((c, True))
        m_lane, s_lane, t_lane = process(chunks)
        m_row = jnp.max(m_lane, axis=1, keepdims=True)
        s_row = jnp.sum(
            s_lane * jnp.exp2((m_lane - m_row) * _LOG2E),
            axis=1,
            keepdims=True,
        )
        t_row = jnp.sum(t_lane, axis=1, keepdims=True)
        nll = m_row + _LN2 * jnp.log2(s_row) - t_row
        loss = jnp.where(tgt == _IGNORE_INDEX, 0.0, nll)
        pt = jnp.exp(-loss)
        fl = _ALPHA * (1.0 - pt) * (1.0 - pt) * loss
        partial = jnp.sum(fl) * (1.0 / _ROWS)

        @pl.when(i == 0)
        def _zero():
            out_ref[0, 0] = 0.0

        out_ref[0, 0] += partial


def kernel(input, target):
    xb = input.astype(jnp.bfloat16)
    hi = jax.lax.bitcast_convert_type(xb[:, :_PCOLS], jnp.uint16)
    lo = jax.lax.bitcast_convert_type(xb[:, _PCOLS:], jnp.uint16)
    packed_bits = (hi.astype(jnp.uint32) << 16) | lo.astype(jnp.uint32)
    packed = jax.lax.bitcast_convert_type(packed_bits, jnp.float32)

    tgt2d = target.astype(jnp.int32).reshape(_ROWS, 1)
    out = pl.pallas_call(
        _focal_kernel,
        grid=(_ROWS // _RBLK, _NJ),
        in_specs=[
            pl.BlockSpec((_RBLK, _CSUB), lambda i, j: (i, j)),
            pl.BlockSpec((_RBLK, 1), lambda i, j: (i, 0)),
        ],
        out_specs=pl.BlockSpec(
            (1, 1), lambda i, j: (0, 0), memory_space=pltpu.SMEM
        ),
        out_shape=jax.ShapeDtypeStruct((1, 1), jnp.float32),
        scratch_shapes=[
            pltpu.VMEM((_RBLK, 128), jnp.float32),
            pltpu.VMEM((_RBLK, 128), jnp.float32),
            pltpu.VMEM((_RBLK, 128), jnp.float32),
        ],
    )(packed, tgt2d)
    return out[0, 0]


# speedup vs baseline: 4.0614x; 1.0181x over previous
import jax
import jax.numpy as jnp
from jax.experimental import pallas as pl
from jax.experimental.pallas import tpu as pltpu

_ALPHA = 0.25
_IGNORE_INDEX = -100

_ROWS = 1024
_COLS = 100000
_RBLK = 256
_CSUB = 4096
_CHUNKS = _CSUB // 128
_NJ = _COLS // _CSUB + 1
_NCOLBLK = (_COLS + _CSUB - 1) // _CSUB

_LOG2E = 1.4426950408889634
_LN2 = 0.6931471805599453


def _focal_kernel(x_ref, tgt_ref, out_ref, m_ref, s_ref, t_ref):
    i = pl.program_id(0)
    j = pl.program_id(1)

    @pl.when(j == 0)
    def _init():
        m_ref[...] = jnp.full((_RBLK, 128), -jnp.inf, jnp.float32)
        s_ref[...] = jnp.zeros((_RBLK, 128), jnp.float32)
        t_ref[...] = jnp.zeros((_RBLK, 128), jnp.float32)

    tgt = tgt_ref[...]
    lane = jax.lax.broadcasted_iota(jnp.int32, (_RBLK, 128), 1)
    rel_tgt = tgt - j * _CSUB
    rel_end = _COLS - j * _CSUB

    rel_tgt_b = jnp.broadcast_to(rel_tgt, (_RBLK, 128))
    rel_end_b = jnp.broadcast_to(jnp.int32(rel_end), (_RBLK, 128))

    def process(chunks):
        m_old = m_ref[...]
        s = s_ref[...]
        t = t_ref[...]
        for g in range(0, len(chunks), 4):
            group = chunks[g:g + 4]
            bms = []
            for c, masked in group:
                xc = x_ref[:, c * 128:(c + 1) * 128]
                if masked:
                    xc = jnp.where(lane + c * 128 < rel_end_b, xc, -jnp.inf)
                t = jnp.where(lane + c * 128 == rel_tgt_b, xc, t)
                bms.append(xc)
            while len(bms) > 1:
                bms = [
                    jnp.maximum(bms[k], bms[k + 1])
                    for k in range(0, len(bms) - 1, 2)
                ] + ([bms[-1]] if len(bms) % 2 else [])
            m_new = jnp.maximum(m_old, bms[0])
            s = s * jnp.exp2((m_old - m_new) * _LOG2E)
            ea = None
            eb = None
            for idx, (c, masked) in enumerate(group):
                xc = x_ref[:, c * 128:(c + 1) * 128]
                e = jnp.exp2((xc - m_new) * _LOG2E)
                if masked:
                    e = jnp.where(lane + c * 128 < rel_end_b, e, 0.0)
                if idx % 2 == 0:
                    ea = e if ea is None else ea + e
                else:
                    eb = e if eb is None else eb + e
            s = s + (ea if eb is None else ea + eb)
            m_old = m_new
        m_ref[...] = m_old
        s_ref[...] = s
        t_ref[...] = t
        return m_old, s, t

    is_last = j == _NJ - 1

    @pl.when(jnp.logical_not(is_last))
    def _full_step():
        process([(c, False) for c in range(_CHUNKS)])

    @pl.when(is_last)
    def _last_step():
        base = (_NJ - 1) * _CSUB
        chunks = []
        for c in range(_CHUNKS):
            start = base + c * 128
            if start + 128 <= _COLS:
                chunks.append((c, False))
            elif start < _COLS:
                chunks.append((c, True))
        m_lane, s_lane, t_lane = process(chunks)
        m_row = jnp.max(m_lane, axis=1, keepdims=True)
        s_row = jnp.sum(
            s_lane * jnp.exp2((m_lane - m_row) * _LOG2E),
            axis=1,
            keepdims=True,
        )
        t_row = jnp.sum(t_lane, axis=1, keepdims=True)
        nll = m_row + _LN2 * jnp.log2(s_row) - t_row
        loss = jnp.where(tgt == _IGNORE_INDEX, 0.0, nll)
        pt = jnp.exp(-loss)
        fl = _ALPHA * (1.0 - pt) * (1.0 - pt) * loss
        partial = jnp.sum(fl) * (1.0 / _ROWS)

        @pl.when(i == 0)
        def _zero():
            out_ref[0, 0] = 0.0

        out_ref[0, 0] += partial


def kernel(input, target):
    tgt2d = target.astype(jnp.int32).reshape(_ROWS, 1)
    out = pl.pallas_call(
        _focal_kernel,
        grid=(_ROWS // _RBLK, _NJ),
        in_specs=[
            pl.BlockSpec((_RBLK, _CSUB), lambda i, j: (i, j)),
            pl.BlockSpec((_RBLK, 1), lambda i, j: (i, 0)),
        ],
        out_specs=pl.BlockSpec(
            (1, 1), lambda i, j: (0, 0), memory_space=pltpu.SMEM
        ),
        out_shape=jax.ShapeDtypeStruct((1, 1), jnp.float32),
        scratch_shapes=[
            pltpu.VMEM((_RBLK, 128), jnp.float32),
            pltpu.VMEM((_RBLK, 128), jnp.float32),
            pltpu.VMEM((_RBLK, 128), jnp.float32),
        ],
    )(input, tgt2d)
    return out[0, 0]


# speedup vs baseline: 4.2481x; 1.0460x over previous
import jax
import jax.numpy as jnp
from jax.experimental import pallas as pl
from jax.experimental.pallas import tpu as pltpu

_ALPHA = 0.25
_IGNORE_INDEX = -100

_ROWS = 1024
_COLS = 100000
_RBLK = 256
_CSUB = 8192
_CHUNKS = _CSUB // 128
_NJ = _COLS // _CSUB + 1
_NCOLBLK = (_COLS + _CSUB - 1) // _CSUB

_LOG2E = 1.4426950408889634
_LN2 = 0.6931471805599453


def _focal_kernel(x_ref, tgt_ref, out_ref, m_ref, s_ref, t_ref):
    i = pl.program_id(0)
    j = pl.program_id(1)

    @pl.when(j == 0)
    def _init():
        m_ref[...] = jnp.full((_RBLK, 128), -jnp.inf, jnp.float32)
        s_ref[...] = jnp.zeros((_RBLK, 128), jnp.float32)
        t_ref[...] = jnp.zeros((_RBLK, 128), jnp.float32)

    tgt = tgt_ref[...]
    lane = jax.lax.broadcasted_iota(jnp.int32, (_RBLK, 128), 1)
    rel_tgt = tgt - j * _CSUB
    rel_end = _COLS - j * _CSUB

    rel_tgt_b = jnp.broadcast_to(rel_tgt, (_RBLK, 128))
    rel_end_b = jnp.broadcast_to(jnp.int32(rel_end), (_RBLK, 128))

    def process(chunks):
        m_old = m_ref[...]
        s = s_ref[...]
        t = t_ref[...]
        for g in range(0, len(chunks), 4):
            group = chunks[g:g + 4]
            bms = []
            for c, masked in group:
                xc = x_ref[:, c * 128:(c + 1) * 128]
                if masked:
                    xc = jnp.where(lane + c * 128 < rel_end_b, xc, -jnp.inf)
                t = jnp.where(lane + c * 128 == rel_tgt_b, xc, t)
                bms.append(xc)
            while len(bms) > 1:
                bms = [
                    jnp.maximum(bms[k], bms[k + 1])
                    for k in range(0, len(bms) - 1, 2)
                ] + ([bms[-1]] if len(bms) % 2 else [])
            m_new = jnp.maximum(m_old, bms[0])
            s = s * jnp.exp2((m_old - m_new) * _LOG2E)
            ea = None
            eb = None
            for idx, (c, masked) in enumerate(group):
                xc = x_ref[:, c * 128:(c + 1) * 128]
                e = jnp.exp2((xc - m_new) * _LOG2E)
                if masked:
                    e = jnp.where(lane + c * 128 < rel_end_b, e, 0.0)
                if idx % 2 == 0:
                    ea = e if ea is None else ea + e
                else:
                    eb = e if eb is None else eb + e
            s = s + (ea if eb is None else ea + eb)
            m_old = m_new
        m_ref[...] = m_old
        s_ref[...] = s
        t_ref[...] = t
        return m_old, s, t

    is_last = j == _NJ - 1

    @pl.when(jnp.logical_not(is_last))
    def _full_step():
        process([(c, False) for c in range(_CHUNKS)])

    @pl.when(is_last)
    def _last_step():
        base = (_NJ - 1) * _CSUB
        chunks = []
        for c in range(_CHUNKS):
            start = base + c * 128
            if start + 128 <= _COLS:
                chunks.append((c, False))
            elif start < _COLS:
                chunks.append((c, True))
        m_lane, s_lane, t_lane = process(chunks)
        m_row = jnp.max(m_lane, axis=1, keepdims=True)
        s_row = jnp.sum(
            s_lane * jnp.exp2((m_lane - m_row) * _LOG2E),
            axis=1,
            keepdims=True,
        )
        t_row = jnp.sum(t_lane, axis=1, keepdims=True)
        nll = m_row + _LN2 * jnp.log2(s_row) - t_row
        loss = jnp.where(tgt == _IGNORE_INDEX, 0.0, nll)
        pt = jnp.exp(-loss)
        fl = _ALPHA * (1.0 - pt) * (1.0 - pt) * loss
        partial = jnp.sum(fl) * (1.0 / _ROWS)

        @pl.when(i == 0)
        def _zero():
            out_ref[0, 0] = 0.0

        out_ref[0, 0] += partial


def kernel(input, target):
    tgt2d = target.astype(jnp.int32).reshape(_ROWS, 1)
    out = pl.pallas_call(
        _focal_kernel,
        grid=(_ROWS // _RBLK, _NJ),
        in_specs=[
            pl.BlockSpec((_RBLK, _CSUB), lambda i, j: (i, j)),
            pl.BlockSpec((_RBLK, 1), lambda i, j: (i, 0)),
        ],
        out_specs=pl.BlockSpec(
            (1, 1), lambda i, j: (0, 0), memory_space=pltpu.SMEM
        ),
        out_shape=jax.ShapeDtypeStruct((1, 1), jnp.float32),
        scratch_shapes=[
            pltpu.VMEM((_RBLK, 128), jnp.float32),
            pltpu.VMEM((_RBLK, 128), jnp.float32),
            pltpu.VMEM((_RBLK, 128), jnp.float32),
        ],
    )(input, tgt2d)
    return out[0, 0]


# speedup vs baseline: 4.2616x; 1.0032x over previous
import jax
import jax.numpy as jnp
from jax.experimental import pallas as pl
from jax.experimental.pallas import tpu as pltpu

_ALPHA = 0.25
_IGNORE_INDEX = -100

_ROWS = 1024
_COLS = 100000
_RBLK = 256
_CSUB = 16384
_CHUNKS = _CSUB // 128
_NJ = _COLS // _CSUB + 1
_NCOLBLK = (_COLS + _CSUB - 1) // _CSUB

_LOG2E = 1.4426950408889634
_LN2 = 0.6931471805599453


def _focal_kernel(x_ref, tgt_ref, out_ref, m_ref, s_ref, t_ref):
    i = pl.program_id(0)
    j = pl.program_id(1)

    @pl.when(j == 0)
    def _init():
        m_ref[...] = jnp.full((_RBLK, 128), -jnp.inf, jnp.float32)
        s_ref[...] = jnp.zeros((_RBLK, 128), jnp.float32)
        t_ref[...] = jnp.zeros((_RBLK, 128), jnp.float32)

    tgt = tgt_ref[...]
    lane = jax.lax.broadcasted_iota(jnp.int32, (_RBLK, 128), 1)
    rel_tgt = tgt - j * _CSUB
    rel_end = _COLS - j * _CSUB

    rel_tgt_b = jnp.broadcast_to(rel_tgt, (_RBLK, 128))
    rel_end_b = jnp.broadcast_to(jnp.int32(rel_end), (_RBLK, 128))

    def process(chunks):
        m_old = m_ref[...]
        s = s_ref[...]
        t = t_ref[...]
        for g in range(0, len(chunks), 4):
            group = chunks[g:g + 4]
            bms = []
            for c, masked in group:
                xc = x_ref[:, c * 128:(c + 1) * 128]
                if masked:
                    xc = jnp.where(lane + c * 128 < rel_end_b, xc, -jnp.inf)
                t = jnp.where(lane + c * 128 == rel_tgt_b, xc, t)
                bms.append(xc)
            while len(bms) > 1:
                bms = [
                    jnp.maximum(bms[k], bms[k + 1])
                    for k in range(0, len(bms) - 1, 2)
                ] + ([bms[-1]] if len(bms) % 2 else [])
            m_new = jnp.maximum(m_old, bms[0])
            s = s * jnp.exp2((m_old - m_new) * _LOG2E)
            ea = None
            eb = None
            for idx, (c, masked) in enumerate(group):
                xc = x_ref[:, c * 128:(c + 1) * 128]
                e = jnp.exp2((xc - m_new) * _LOG2E)
                if masked:
                    e = jnp.where(lane + c * 128 < rel_end_b, e, 0.0)
                if idx % 2 == 0:
                    ea = e if ea is None else ea + e
                else:
                    eb = e if eb is None else eb + e
            s = s + (ea if eb is None else ea + eb)
            m_old = m_new
        m_ref[...] = m_old
        s_ref[...] = s
        t_ref[...] = t
        return m_old, s, t

    is_last = j == _NJ - 1

    @pl.when(jnp.logical_not(is_last))
    def _full_step():
        process([(c, False) for c in range(_CHUNKS)])

    @pl.when(is_last)
    def _last_step():
        base = (_NJ - 1) * _CSUB
        chunks = []
        for c in range(_CHUNKS):
            start = base + c * 128
            if start + 128 <= _COLS:
                chunks.append((c, False))
            elif start < _COLS:
                chunks.append((c, True))
        m_lane, s_lane, t_lane = process(chunks)
        m_row = jnp.max(m_lane, axis=1, keepdims=True)
        s_row = jnp.sum(
            s_lane * jnp.exp2((m_lane - m_row) * _LOG2E),
            axis=1,
            keepdims=True,
        )
        t_row = jnp.sum(t_lane, axis=1, keepdims=True)
        nll = m_row + _LN2 * jnp.log2(s_row) - t_row
        loss = jnp.where(tgt == _IGNORE_INDEX, 0.0, nll)
        pt = jnp.exp(-loss)
        fl = _ALPHA * (1.0 - pt) * (1.0 - pt) * loss
        partial = jnp.sum(fl) * (1.0 / _ROWS)

        @pl.when(i == 0)
        def _zero():
            out_ref[0, 0] = 0.0

        out_ref[0, 0] += partial


def kernel(input, target):
    tgt2d = target.astype(jnp.int32).reshape(_ROWS, 1)
    out = pl.pallas_call(
        _focal_kernel,
        grid=(_ROWS // _RBLK, _NJ),
        in_specs=[
            pl.BlockSpec((_RBLK, _CSUB), lambda i, j: (i, j)),
            pl.BlockSpec((_RBLK, 1), lambda i, j: (i, 0)),
        ],
        out_specs=pl.BlockSpec(
            (1, 1), lambda i, j: (0, 0), memory_space=pltpu.SMEM
        ),
        out_shape=jax.ShapeDtypeStruct((1, 1), jnp.float32),
        scratch_shapes=[
            pltpu.VMEM((_RBLK, 128), jnp.float32),
            pltpu.VMEM((_RBLK, 128), jnp.float32),
            pltpu.VMEM((_RBLK, 128), jnp.float32),
        ],
    )(input, tgt2d)
    return out[0, 0]
